# Initial kernel scaffold; baseline (speedup 1.0000x reference)
import jax
import jax.numpy as jnp
from jax.experimental import pallas as pl


def kernel(ts_emb_seq, codebook):
    raise NotImplementedError("write your pallas kernel here")


# speedup vs baseline: 2.0453x; 2.0453x over previous
import functools

import jax
import jax.numpy as jnp
from jax.experimental import pallas as pl
from jax.experimental.pallas import tpu as pltpu


def _som_tile(z_ref, cbt_ref, cb_ref, zq_ref, q_ref, bmu_ref, *, alpha, k):
    z = z_ref[...]
    cbt = cbt_ref[...]
    cb = cb_ref[...]

    z_sq = jnp.sum(z * z, axis=1, keepdims=True)
    c_sq = jnp.sum(cbt * cbt, axis=0, keepdims=True)
    cross = jnp.dot(z, cbt, preferred_element_type=jnp.float32)
    d = jnp.maximum(z_sq - 2.0 * cross + c_sq, 0.0)

    t = 1.0 + d * (1.0 / alpha)
    u = 1.0 / (t * t * t)
    q = u * (1.0 / jnp.sum(u, axis=1, keepdims=True))
    q_ref[...] = q

    d_min = jnp.min(d, axis=1, keepdims=True)
    iota_k = jax.lax.broadcasted_iota(jnp.int32, d.shape, 1)
    masked = jnp.where(d == d_min, iota_k, k)
    bmu = jnp.min(masked, axis=1)
    bmu_ref[...] = bmu

    one_hot = (iota_k == bmu[:, None]).astype(jnp.float32)
    zq_ref[...] = jnp.dot(one_hot, cb, preferred_element_type=jnp.float32)


def kernel(ts_emb_seq, codebook):
    alpha = 5.0
    b, t_max, d_latent = ts_emb_seq.shape
    n = b * t_max
    k = codebook.shape[0]
    z = ts_emb_seq.reshape(n, d_latent)
    cbt = codebook.T

    tn = 512
    grid = (n // tn,)

    zq, q, bmu = pl.pallas_call(
        functools.partial(_som_tile, alpha=alpha, k=k),
        grid=grid,
        in_specs=[
            pl.BlockSpec((tn, d_latent), lambda i: (i, 0)),
            pl.BlockSpec((d_latent, k), lambda i: (0, 0)),
            pl.BlockSpec((k, d_latent), lambda i: (0, 0)),
        ],
        out_specs=[
            pl.BlockSpec((tn, d_latent), lambda i: (i, 0)),
            pl.BlockSpec((tn, k), lambda i: (i, 0)),
            pl.BlockSpec((tn,), lambda i: (i,)),
        ],
        out_shape=[
            jax.ShapeDtypeStruct((n, d_latent), jnp.float32),
            jax.ShapeDtypeStruct((n, k), jnp.float32),
            jax.ShapeDtypeStruct((n,), jnp.int32),
        ],
    )(z, cbt, codebook)

    return zq, q, q, bmu


# speedup vs baseline: 2.9002x; 1.4180x over previous
import functools

import jax
import jax.numpy as jnp
from jax.experimental import pallas as pl
from jax.experimental.pallas import tpu as pltpu


def _som_tile(z_ref, cbt_ref, cb_ref, zq_ref, q_ref, q2_ref, bmu_ref, *, alpha, k):
    z = z_ref[...]
    cbt = cbt_ref[...]
    cb = cb_ref[...]

    z_sq = jnp.sum(z * z, axis=1, keepdims=True)
    c_sq = jnp.sum(cbt * cbt, axis=0, keepdims=True)
    cross = jnp.dot(z, cbt, preferred_element_type=jnp.float32)
    d = jnp.maximum(z_sq - 2.0 * cross + c_sq, 0.0)

    t = 1.0 + d * (1.0 / alpha)
    u = 1.0 / (t * t * t)
    q = u * (1.0 / jnp.sum(u, axis=1, keepdims=True))
    q_ref[...] = q
    q2_ref[...] = q

    d_min = jnp.min(d, axis=1, keepdims=True)
    iota_k = jax.lax.broadcasted_iota(jnp.int32, d.shape, 1)
    masked = jnp.where(d == d_min, iota_k, k)
    bmu = jnp.min(masked, axis=1)
    bmu_ref[...] = bmu

    one_hot = (iota_k == bmu[:, None]).astype(jnp.float32)
    zq_ref[...] = jnp.dot(one_hot, cb, preferred_element_type=jnp.float32)


def kernel(ts_emb_seq, codebook):
    alpha = 5.0
    b, t_max, d_latent = ts_emb_seq.shape
    n = b * t_max
    k = codebook.shape[0]
    z = ts_emb_seq.reshape(n, d_latent)
    cbt = codebook.T

    tn = 512
    grid = (n // tn,)

    zq, q, q2, bmu = pl.pallas_call(
        functools.partial(_som_tile, alpha=alpha, k=k),
        grid=grid,
        in_specs=[
            pl.BlockSpec((tn, d_latent), lambda i: (i, 0)),
            pl.BlockSpec((d_latent, k), lambda i: (0, 0)),
            pl.BlockSpec((k, d_latent), lambda i: (0, 0)),
        ],
        out_specs=[
            pl.BlockSpec((tn, d_latent), lambda i: (i, 0)),
            pl.BlockSpec((tn, k), lambda i: (i, 0)),
            pl.BlockSpec((tn, k), lambda i: (i, 0)),
            pl.BlockSpec((tn,), lambda i: (i,)),
        ],
        out_shape=[
            jax.ShapeDtypeStruct((n, d_latent), jnp.float32),
            jax.ShapeDtypeStruct((n, k), jnp.float32),
            jax.ShapeDtypeStruct((n, k), jnp.float32),
            jax.ShapeDtypeStruct((n,), jnp.int32),
        ],
    )(z, cbt, codebook)

    return zq, q, q2, bmu
